# baseline (speedup 1.0000x reference)
import functools

import jax
import jax.numpy as jnp
from jax import lax
from jax.experimental import pallas as pl
from jax.experimental.pallas import tpu as pltpu
from jax.experimental.pallas import tpu_sc as plsc

N_NODES = 10000
N_EDGES = 320000
NCLASS = 40

NC, NS = 2, 16
NW = NC * NS
BATCH = 128
NBATCH = N_EDGES // BATCH
NB_MAIN = NBATCH // NW
NB_EXTRA = NBATCH - NB_MAIN * NW
ROWS_A = 640
ROWS_B = N_NODES - (NS - 1) * ROWS_A


def _make_seg_sum(F):
  mesh = plsc.VectorSubcoreMesh(core_axis_name="c", subcore_axis_name="s",
                                num_cores=NC, num_subcores=NS)

  @functools.partial(
      pl.kernel,
      out_type=jax.ShapeDtypeStruct((NC, N_NODES, F), jnp.float32),
      mesh=mesh,
      compiler_params=pltpu.CompilerParams(use_tc_tiling_on_sc=(F % 128 == 0)),
      scratch_types=[
          pltpu.VMEM((4, 2, BATCH), jnp.int32),
          pltpu.VMEM((BATCH, F), jnp.float32),
          pltpu.VMEM((BATCH, F), jnp.float32),
          pltpu.VMEM((BATCH, F), jnp.float32),
          pltpu.VMEM_SHARED((N_NODES, F), jnp.float32),
          pltpu.SemaphoreType.DMA,
          pltpu.SemaphoreType.DMA,
          pltpu.SemaphoreType.DMA,
      ],
  )
  def seg_sum(x_hbm, ei_hbm, zeros_hbm, out_hbm,
              idxr, rows0, rows1, rows2, acc,
              isem, gsem, ssem):
    c = lax.axis_index("c")
    s = lax.axis_index("s")
    wid = c * NS + s
    stripe_off = pl.multiple_of(s * ROWS_A, 8)
    rows = (rows0, rows1, rows2)

    def start_idx(j, b4):
      off = pl.multiple_of((j * NW + wid) * BATCH, BATCH)
      pltpu.async_copy(ei_hbm.at[:, pl.ds(off, BATCH)], idxr.at[b4], isem)

    def wait_idx(b4):
      pltpu.make_async_copy(ei_hbm.at[:, pl.ds(0, BATCH)], idxr.at[b4],
                            isem).wait()

    def start_gather(b3, b4):
      pltpu.async_copy(x_hbm.at[idxr.at[b4, 0]], rows[b3], gsem)

    def wait_gather(b3, b4):
      pltpu.make_async_copy(x_hbm.at[idxr.at[b4, 0]], rows[b3], gsem).wait()

    def start_scatter(b3, b4):
      pltpu.async_copy(rows[b3], acc.at[idxr.at[b4, 1]], ssem, add=True)

    def wait_scatter(b3, b4):
      pltpu.make_async_copy(rows[b3], acc.at[idxr.at[b4, 1]], ssem).wait()

    start_idx(0, 0)

    @pl.when(s < NS - 1)
    def _():
      pltpu.sync_copy(zeros_hbm.at[pl.ds(stripe_off, ROWS_A)],
                      acc.at[pl.ds(stripe_off, ROWS_A)])

    @pl.when(s == NS - 1)
    def _():
      pltpu.sync_copy(zeros_hbm.at[pl.ds((NS - 1) * ROWS_A, ROWS_B)],
                      acc.at[pl.ds((NS - 1) * ROWS_A, ROWS_B)])

    plsc.subcore_barrier()

    def body_steady(j, prefetch):
      wait_scatter((j - 3) % 3, (j - 3) % 4)
      if prefetch:
        start_idx(j + 1, (j + 1) % 4)
      wait_idx(j % 4)
      start_gather(j % 3, j % 4)
      wait_gather((j - 1) % 3, (j - 1) % 4)
      start_scatter((j - 1) % 3, (j - 1) % 4)

    wait_idx(0)
    start_gather(0, 0)
    start_idx(1, 1)
    wait_idx(1)
    start_gather(1, 1)
    start_idx(2, 2)
    wait_gather(0, 0)
    start_scatter(0, 0)
    wait_idx(2)
    start_gather(2, 2)
    start_idx(3, 3)
    wait_gather(1, 1)
    start_scatter(1, 1)

    def body_steady_static(t, j_dyn):
      ts = t + 3
      wait_scatter((ts - 3) % 3, (ts - 3) % 4)
      start_idx(j_dyn + 1, (ts + 1) % 4)
      wait_idx(ts % 4)
      start_gather(ts % 3, ts % 4)
      wait_gather((ts - 1) % 3, (ts - 1) % 4)
      start_scatter((ts - 1) % 3, (ts - 1) % 4)

    def body(g, carry):
      for t in range(12):
        j = 12 * g + 3 + t
        body_steady_static(t, j)
      return carry

    lax.fori_loop(0, (NB_MAIN - 6) // 12, body, 0)

    for j in range(NB_MAIN - 3, NB_MAIN):
      body_steady(j, j + 1 <= NB_MAIN - 1)

    wait_gather((NB_MAIN - 1) % 3, (NB_MAIN - 1) % 4)
    start_scatter((NB_MAIN - 1) % 3, (NB_MAIN - 1) % 4)
    wait_scatter((NB_MAIN - 3) % 3, (NB_MAIN - 3) % 4)
    wait_scatter((NB_MAIN - 2) % 3, (NB_MAIN - 2) % 4)
    wait_scatter((NB_MAIN - 1) % 3, (NB_MAIN - 1) % 4)

    @pl.when(wid < NB_EXTRA)
    def _():
      eoff = pl.multiple_of((NB_MAIN * NW + wid) * BATCH, BATCH)
      pltpu.sync_copy(ei_hbm.at[:, pl.ds(eoff, BATCH)], idxr.at[0])
      pltpu.async_copy(x_hbm.at[idxr.at[0, 0]], rows[0], gsem).wait()
      pltpu.async_copy(rows[0], acc.at[idxr.at[0, 1]], ssem, add=True).wait()

    plsc.subcore_barrier()

    @pl.when(s < NS - 1)
    def _():
      pltpu.sync_copy(acc.at[pl.ds(stripe_off, ROWS_A)],
                      out_hbm.at[c, pl.ds(stripe_off, ROWS_A)])

    @pl.when(s == NS - 1)
    def _():
      pltpu.sync_copy(acc.at[pl.ds((NS - 1) * ROWS_A, ROWS_B)],
                      out_hbm.at[c, pl.ds((NS - 1) * ROWS_A, ROWS_B)])

  return seg_sum


_make_seg_sum = functools.lru_cache(None)(_make_seg_sum)

_BLK = 1000


def _tc_base_body(x_ref, ws_ref, b_ref, o_ref):
  o_ref[...] = (jnp.dot(x_ref[...], ws_ref[...],
                        preferred_element_type=jnp.float32) + b_ref[...])


def _tc_base(x, Ws, b):
  n, f_in = x.shape
  f_out = Ws.shape[1]
  grid = n // _BLK
  return pl.pallas_call(
      _tc_base_body,
      grid=(grid,),
      in_specs=[
          pl.BlockSpec((_BLK, f_in), lambda i: (i, 0)),
          pl.BlockSpec((f_in, f_out), lambda i: (0, 0)),
          pl.BlockSpec((1, f_out), lambda i: (0, 0)),
      ],
      out_specs=pl.BlockSpec((_BLK, f_out), lambda i: (i, 0)),
      out_shape=jax.ShapeDtypeStruct((n, f_out), jnp.float32),
  )(x, Ws, b.reshape(1, f_out))


def _tc_comb_body(p_ref, base_ref, w_ref, o_ref):
  agg = p_ref[0] + p_ref[1]
  o_ref[...] = (jnp.dot(agg, w_ref[...], preferred_element_type=jnp.float32)
                + base_ref[...])


def _tc_comb(p, base, W):
  n, f_out = base.shape
  f_in = W.shape[0]
  grid = n // _BLK
  return pl.pallas_call(
      _tc_comb_body,
      grid=(grid,),
      in_specs=[
          pl.BlockSpec((NC, _BLK, f_in), lambda i: (0, i, 0)),
          pl.BlockSpec((_BLK, f_out), lambda i: (i, 0)),
          pl.BlockSpec((f_in, f_out), lambda i: (0, 0)),
      ],
      out_specs=pl.BlockSpec((_BLK, f_out), lambda i: (i, 0)),
      out_shape=jax.ShapeDtypeStruct((n, f_out), jnp.float32),
  )(p, base, W)


def _tc_comb2_body(p_ref, base_ref, w_ref, wo_ref, wso_ref, bo_ref,
                   sup_ref, base3_ref):
  agg = p_ref[0] + p_ref[1]
  x2 = (jnp.dot(agg, w_ref[...], preferred_element_type=jnp.float32)
        + base_ref[...])
  sup_ref[...] = jnp.dot(x2, wo_ref[...], preferred_element_type=jnp.float32)
  base3_ref[...] = (jnp.dot(x2, wso_ref[...],
                            preferred_element_type=jnp.float32) + bo_ref[...])


def _tc_comb2(p, base, W, Wo, Wso, bo):
  n, f_mid = base.shape
  f_out = Wo.shape[1]
  grid = n // _BLK
  return pl.pallas_call(
      _tc_comb2_body,
      grid=(grid,),
      in_specs=[
          pl.BlockSpec((NC, _BLK, f_mid), lambda i: (0, i, 0)),
          pl.BlockSpec((_BLK, f_mid), lambda i: (i, 0)),
          pl.BlockSpec((f_mid, f_mid), lambda i: (0, 0)),
          pl.BlockSpec((f_mid, f_out), lambda i: (0, 0)),
          pl.BlockSpec((f_mid, f_out), lambda i: (0, 0)),
          pl.BlockSpec((1, f_out), lambda i: (0, 0)),
      ],
      out_specs=[
          pl.BlockSpec((_BLK, f_out), lambda i: (i, 0)),
          pl.BlockSpec((_BLK, f_out), lambda i: (i, 0)),
      ],
      out_shape=[
          jax.ShapeDtypeStruct((n, f_out), jnp.float32),
          jax.ShapeDtypeStruct((n, f_out), jnp.float32),
      ],
  )(p, base, W, Wo, Wso, bo.reshape(1, f_out))


def _tc_final_body(p_ref, base_ref, o_ref):
  z = p_ref[0] + p_ref[1] + base_ref[...]
  m = jnp.max(z, axis=1, keepdims=True)
  zs = z - m
  o_ref[...] = zs - jnp.log(jnp.sum(jnp.exp(zs), axis=1, keepdims=True))


def _tc_final(p, base):
  n, f_out = base.shape
  grid = n // _BLK
  return pl.pallas_call(
      _tc_final_body,
      grid=(grid,),
      in_specs=[
          pl.BlockSpec((NC, _BLK, f_out), lambda i: (0, i, 0)),
          pl.BlockSpec((_BLK, f_out), lambda i: (i, 0)),
      ],
      out_specs=pl.BlockSpec((_BLK, f_out), lambda i: (i, 0)),
      out_shape=jax.ShapeDtypeStruct((n, f_out), jnp.float32),
  )(p, base)


def kernel(fea, edge_index, W_in, Ws_in, b_in, W_mid, Ws_mid, b_mid,
           W_out, Ws_out, b_out):
  zeros128 = jnp.zeros((N_NODES, 128), jnp.float32)
  zeros40 = jnp.zeros((N_NODES, NCLASS), jnp.float32)
  seg_sum_128 = _make_seg_sum(128)
  seg_sum_40 = _make_seg_sum(NCLASS)

  base1 = _tc_base(fea, Ws_in, b_in)
  p1 = seg_sum_128(fea, edge_index, zeros128)
  x1 = _tc_comb(p1, base1, W_in)
  base2 = _tc_base(x1, Ws_mid, b_mid)
  p2 = seg_sum_128(x1, edge_index, zeros128)
  sup3, base3 = _tc_comb2(p2, base2, W_mid, W_out, Ws_out, b_out)
  p3 = seg_sum_40(sup3, edge_index, zeros40)
  return _tc_final(p3, base3)


# speedup vs baseline: 1.0262x; 1.0262x over previous
import functools

import jax
import jax.numpy as jnp
from jax import lax
from jax.experimental import pallas as pl
from jax.experimental.pallas import tpu as pltpu
from jax.experimental.pallas import tpu_sc as plsc

N_NODES = 10000
N_EDGES = 320000
NCLASS = 40

NC, NS = 2, 16
NW = NC * NS
BATCH = 128
NBATCH = N_EDGES // BATCH
NB_MAIN = NBATCH // NW
NB_EXTRA = NBATCH - NB_MAIN * NW
ROWS_A = 640
ROWS_B = N_NODES - (NS - 1) * ROWS_A


def _make_seg_sum(F):
  mesh = plsc.VectorSubcoreMesh(core_axis_name="c", subcore_axis_name="s",
                                num_cores=NC, num_subcores=NS)

  @functools.partial(
      pl.kernel,
      out_type=jax.ShapeDtypeStruct((NC, N_NODES, F), jnp.float32),
      mesh=mesh,
      compiler_params=pltpu.CompilerParams(use_tc_tiling_on_sc=(F % 128 == 0)),
      scratch_types=[
          pltpu.VMEM((4, 2, BATCH), jnp.int32),
          pltpu.VMEM((BATCH, F), jnp.float32),
          pltpu.VMEM((BATCH, F), jnp.float32),
          pltpu.VMEM((BATCH, F), jnp.float32),
          pltpu.VMEM_SHARED((N_NODES, F), jnp.float32),
          pltpu.SemaphoreType.DMA,
          pltpu.SemaphoreType.DMA,
          pltpu.SemaphoreType.DMA,
      ],
  )
  def seg_sum(x_hbm, ei_hbm, zeros_hbm, out_hbm,
              idxr, rows0, rows1, rows2, acc,
              isem, gsem, ssem):
    c = lax.axis_index("c")
    s = lax.axis_index("s")
    wid = c * NS + s
    stripe_off = pl.multiple_of(s * ROWS_A, 8)
    rows = (rows0, rows1, rows2)

    def start_idx(j, b4):
      off = pl.multiple_of((j * NW + wid) * BATCH, BATCH)
      pltpu.async_copy(ei_hbm.at[:, pl.ds(off, BATCH)], idxr.at[b4], isem)

    def wait_idx(b4):
      pltpu.make_async_copy(ei_hbm.at[:, pl.ds(0, BATCH)], idxr.at[b4],
                            isem).wait()

    def start_gather(b3, b4):
      pltpu.async_copy(x_hbm.at[idxr.at[b4, 0]], rows[b3], gsem)

    def wait_gather(b3, b4):
      pltpu.make_async_copy(x_hbm.at[idxr.at[b4, 0]], rows[b3], gsem).wait()

    def start_scatter(b3, b4):
      pltpu.async_copy(rows[b3], acc.at[idxr.at[b4, 1]], ssem, add=True)

    def wait_scatter(b3, b4):
      pltpu.make_async_copy(rows[b3], acc.at[idxr.at[b4, 1]], ssem).wait()

    start_idx(0, 0)

    @pl.when(s < NS - 1)
    def _():
      pltpu.async_copy(zeros_hbm.at[pl.ds(stripe_off, ROWS_A)],
                       acc.at[pl.ds(stripe_off, ROWS_A)], ssem)

    @pl.when(s == NS - 1)
    def _():
      pltpu.async_copy(zeros_hbm.at[pl.ds((NS - 1) * ROWS_A, ROWS_B)],
                       acc.at[pl.ds((NS - 1) * ROWS_A, ROWS_B)], ssem)

    def body_steady(j, prefetch):
      wait_scatter((j - 3) % 3, (j - 3) % 4)
      if prefetch:
        start_idx(j + 1, (j + 1) % 4)
      wait_idx(j % 4)
      start_gather(j % 3, j % 4)
      wait_gather((j - 1) % 3, (j - 1) % 4)
      start_scatter((j - 1) % 3, (j - 1) % 4)

    wait_idx(0)
    start_gather(0, 0)
    start_idx(1, 1)
    wait_idx(1)
    start_gather(1, 1)
    start_idx(2, 2)

    @pl.when(s < NS - 1)
    def _():
      pltpu.make_async_copy(zeros_hbm.at[pl.ds(stripe_off, ROWS_A)],
                            acc.at[pl.ds(stripe_off, ROWS_A)], ssem).wait()

    @pl.when(s == NS - 1)
    def _():
      pltpu.make_async_copy(zeros_hbm.at[pl.ds((NS - 1) * ROWS_A, ROWS_B)],
                            acc.at[pl.ds((NS - 1) * ROWS_A, ROWS_B)],
                            ssem).wait()

    plsc.subcore_barrier()

    wait_gather(0, 0)
    start_scatter(0, 0)
    wait_idx(2)
    start_gather(2, 2)
    start_idx(3, 3)
    wait_gather(1, 1)
    start_scatter(1, 1)

    def body_steady_static(t, j_dyn):
      ts = t + 3
      wait_scatter((ts - 3) % 3, (ts - 3) % 4)
      start_idx(j_dyn + 1, (ts + 1) % 4)
      wait_idx(ts % 4)
      start_gather(ts % 3, ts % 4)
      wait_gather((ts - 1) % 3, (ts - 1) % 4)
      start_scatter((ts - 1) % 3, (ts - 1) % 4)

    def body(g, carry):
      for t in range(12):
        j = 12 * g + 3 + t
        body_steady_static(t, j)
      return carry

    lax.fori_loop(0, (NB_MAIN - 6) // 12, body, 0)

    for j in range(NB_MAIN - 3, NB_MAIN):
      body_steady(j, j + 1 <= NB_MAIN - 1)

    wait_gather((NB_MAIN - 1) % 3, (NB_MAIN - 1) % 4)
    start_scatter((NB_MAIN - 1) % 3, (NB_MAIN - 1) % 4)
    wait_scatter((NB_MAIN - 3) % 3, (NB_MAIN - 3) % 4)
    wait_scatter((NB_MAIN - 2) % 3, (NB_MAIN - 2) % 4)
    wait_scatter((NB_MAIN - 1) % 3, (NB_MAIN - 1) % 4)

    @pl.when(wid < NB_EXTRA)
    def _():
      eoff = pl.multiple_of((NB_MAIN * NW + wid) * BATCH, BATCH)
      pltpu.sync_copy(ei_hbm.at[:, pl.ds(eoff, BATCH)], idxr.at[0])
      pltpu.async_copy(x_hbm.at[idxr.at[0, 0]], rows[0], gsem).wait()
      pltpu.async_copy(rows[0], acc.at[idxr.at[0, 1]], ssem, add=True).wait()

    plsc.subcore_barrier()

    @pl.when(s < NS - 1)
    def _():
      pltpu.sync_copy(acc.at[pl.ds(stripe_off, ROWS_A)],
                      out_hbm.at[c, pl.ds(stripe_off, ROWS_A)])

    @pl.when(s == NS - 1)
    def _():
      pltpu.sync_copy(acc.at[pl.ds((NS - 1) * ROWS_A, ROWS_B)],
                      out_hbm.at[c, pl.ds((NS - 1) * ROWS_A, ROWS_B)])

  return seg_sum


_make_seg_sum = functools.lru_cache(None)(_make_seg_sum)

_BLK = 1000


def _tc_base_body(x_ref, ws_ref, b_ref, o_ref):
  o_ref[...] = (jnp.dot(x_ref[...], ws_ref[...],
                        preferred_element_type=jnp.float32) + b_ref[...])


def _tc_base(x, Ws, b):
  n, f_in = x.shape
  f_out = Ws.shape[1]
  grid = n // _BLK
  return pl.pallas_call(
      _tc_base_body,
      grid=(grid,),
      in_specs=[
          pl.BlockSpec((_BLK, f_in), lambda i: (i, 0)),
          pl.BlockSpec((f_in, f_out), lambda i: (0, 0)),
          pl.BlockSpec((1, f_out), lambda i: (0, 0)),
      ],
      out_specs=pl.BlockSpec((_BLK, f_out), lambda i: (i, 0)),
      out_shape=jax.ShapeDtypeStruct((n, f_out), jnp.float32),
  )(x, Ws, b.reshape(1, f_out))


def _tc_layer_body(p_ref, x_ref, w_ref, ws_ref, b_ref, o_ref):
  agg = p_ref[0] + p_ref[1]
  o_ref[...] = (jnp.dot(agg, w_ref[...], preferred_element_type=jnp.float32)
                + jnp.dot(x_ref[...], ws_ref[...],
                          preferred_element_type=jnp.float32)
                + b_ref[...])


def _tc_layer(p, x, W, Ws, b):
  n, f_in = x.shape
  f_out = W.shape[1]
  grid = n // _BLK
  return pl.pallas_call(
      _tc_layer_body,
      grid=(grid,),
      in_specs=[
          pl.BlockSpec((NC, _BLK, f_in), lambda i: (0, i, 0)),
          pl.BlockSpec((_BLK, f_in), lambda i: (i, 0)),
          pl.BlockSpec((f_in, f_out), lambda i: (0, 0)),
          pl.BlockSpec((f_in, f_out), lambda i: (0, 0)),
          pl.BlockSpec((1, f_out), lambda i: (0, 0)),
      ],
      out_specs=pl.BlockSpec((_BLK, f_out), lambda i: (i, 0)),
      out_shape=jax.ShapeDtypeStruct((n, f_out), jnp.float32),
  )(p, x, W, Ws, b.reshape(1, f_out))


def _tc_layer2_body(p_ref, x_ref, w_ref, ws_ref, b_ref,
                    wo_ref, wso_ref, bo_ref, sup_ref, base_ref):
  agg = p_ref[0] + p_ref[1]
  x2 = (jnp.dot(agg, w_ref[...], preferred_element_type=jnp.float32)
        + jnp.dot(x_ref[...], ws_ref[...], preferred_element_type=jnp.float32)
        + b_ref[...])
  sup_ref[...] = jnp.dot(x2, wo_ref[...], preferred_element_type=jnp.float32)
  base_ref[...] = (jnp.dot(x2, wso_ref[...],
                           preferred_element_type=jnp.float32) + bo_ref[...])


def _tc_layer2(p, x, W, Ws, b, Wo, Wso, bo):
  n, f_in = x.shape
  f_mid = W.shape[1]
  f_out = Wo.shape[1]
  grid = n // _BLK
  return pl.pallas_call(
      _tc_layer2_body,
      grid=(grid,),
      in_specs=[
          pl.BlockSpec((NC, _BLK, f_in), lambda i: (0, i, 0)),
          pl.BlockSpec((_BLK, f_in), lambda i: (i, 0)),
          pl.BlockSpec((f_in, f_mid), lambda i: (0, 0)),
          pl.BlockSpec((f_in, f_mid), lambda i: (0, 0)),
          pl.BlockSpec((1, f_mid), lambda i: (0, 0)),
          pl.BlockSpec((f_mid, f_out), lambda i: (0, 0)),
          pl.BlockSpec((f_mid, f_out), lambda i: (0, 0)),
          pl.BlockSpec((1, f_out), lambda i: (0, 0)),
      ],
      out_specs=[
          pl.BlockSpec((_BLK, f_out), lambda i: (i, 0)),
          pl.BlockSpec((_BLK, f_out), lambda i: (i, 0)),
      ],
      out_shape=[
          jax.ShapeDtypeStruct((n, f_out), jnp.float32),
          jax.ShapeDtypeStruct((n, f_out), jnp.float32),
      ],
  )(p, x, W, Ws, b.reshape(1, f_mid), Wo, Wso, bo.reshape(1, f_out))


def _tc_comb_body(p_ref, base_ref, w_ref, o_ref):
  agg = p_ref[0] + p_ref[1]
  o_ref[...] = (jnp.dot(agg, w_ref[...], preferred_element_type=jnp.float32)
                + base_ref[...])


def _tc_comb(p, base, W):
  n, f_out = base.shape
  f_in = W.shape[0]
  grid = n // _BLK
  return pl.pallas_call(
      _tc_comb_body,
      grid=(grid,),
      in_specs=[
          pl.BlockSpec((NC, _BLK, f_in), lambda i: (0, i, 0)),
          pl.BlockSpec((_BLK, f_out), lambda i: (i, 0)),
          pl.BlockSpec((f_in, f_out), lambda i: (0, 0)),
      ],
      out_specs=pl.BlockSpec((_BLK, f_out), lambda i: (i, 0)),
      out_shape=jax.ShapeDtypeStruct((n, f_out), jnp.float32),
  )(p, base, W)


def _tc_comb2_body(p_ref, base_ref, w_ref, wo_ref, wso_ref, bo_ref,
                   sup_ref, base3_ref):
  agg = p_ref[0] + p_ref[1]
  x2 = (jnp.dot(agg, w_ref[...], preferred_element_type=jnp.float32)
        + base_ref[...])
  sup_ref[...] = jnp.dot(x2, wo_ref[...], preferred_element_type=jnp.float32)
  base3_ref[...] = (jnp.dot(x2, wso_ref[...],
                            preferred_element_type=jnp.float32) + bo_ref[...])


def _tc_comb2(p, base, W, Wo, Wso, bo):
  n, f_mid = base.shape
  f_out = Wo.shape[1]
  grid = n // _BLK
  return pl.pallas_call(
      _tc_comb2_body,
      grid=(grid,),
      in_specs=[
          pl.BlockSpec((NC, _BLK, f_mid), lambda i: (0, i, 0)),
          pl.BlockSpec((_BLK, f_mid), lambda i: (i, 0)),
          pl.BlockSpec((f_mid, f_mid), lambda i: (0, 0)),
          pl.BlockSpec((f_mid, f_out), lambda i: (0, 0)),
          pl.BlockSpec((f_mid, f_out), lambda i: (0, 0)),
          pl.BlockSpec((1, f_out), lambda i: (0, 0)),
      ],
      out_specs=[
          pl.BlockSpec((_BLK, f_out), lambda i: (i, 0)),
          pl.BlockSpec((_BLK, f_out), lambda i: (i, 0)),
      ],
      out_shape=[
          jax.ShapeDtypeStruct((n, f_out), jnp.float32),
          jax.ShapeDtypeStruct((n, f_out), jnp.float32),
      ],
  )(p, base, W, Wo, Wso, bo.reshape(1, f_out))


def _tc_final_body(p_ref, base_ref, o_ref):
  z = p_ref[0] + p_ref[1] + base_ref[...]
  m = jnp.max(z, axis=1, keepdims=True)
  zs = z - m
  o_ref[...] = zs - jnp.log(jnp.sum(jnp.exp(zs), axis=1, keepdims=True))


def _tc_final(p, base):
  n, f_out = base.shape
  grid = n // _BLK
  return pl.pallas_call(
      _tc_final_body,
      grid=(grid,),
      in_specs=[
          pl.BlockSpec((NC, _BLK, f_out), lambda i: (0, i, 0)),
          pl.BlockSpec((_BLK, f_out), lambda i: (i, 0)),
      ],
      out_specs=pl.BlockSpec((_BLK, f_out), lambda i: (i, 0)),
      out_shape=jax.ShapeDtypeStruct((n, f_out), jnp.float32),
  )(p, base)


def kernel(fea, edge_index, W_in, Ws_in, b_in, W_mid, Ws_mid, b_mid,
           W_out, Ws_out, b_out):
  zeros128 = jnp.zeros((N_NODES, 128), jnp.float32)
  zeros40 = jnp.zeros((N_NODES, NCLASS), jnp.float32)
  seg_sum_128 = _make_seg_sum(128)
  seg_sum_40 = _make_seg_sum(NCLASS)

  p1 = seg_sum_128(fea, edge_index, zeros128)
  x1 = _tc_layer(p1, fea, W_in, Ws_in, b_in)
  p2 = seg_sum_128(x1, edge_index, zeros128)
  sup3, base3 = _tc_layer2(p2, x1, W_mid, Ws_mid, b_mid, W_out, Ws_out, b_out)
  p3 = seg_sum_40(sup3, edge_index, zeros40)
  return _tc_final(p3, base3)


# speedup vs baseline: 1.0500x; 1.0232x over previous
import functools

import jax
import jax.numpy as jnp
from jax import lax
from jax.experimental import pallas as pl
from jax.experimental.pallas import tpu as pltpu
from jax.experimental.pallas import tpu_sc as plsc

N_NODES = 10000
N_EDGES = 320000
NCLASS = 40

NC, NS = 2, 16
NW = NC * NS
BATCH = 128
NBATCH = N_EDGES // BATCH
NB_MAIN = NBATCH // NW
NB_EXTRA = NBATCH - NB_MAIN * NW
ROWS_A = 640
ROWS_B = N_NODES - (NS - 1) * ROWS_A


def _make_seg_sum(F):
  mesh = plsc.VectorSubcoreMesh(core_axis_name="c", subcore_axis_name="s",
                                num_cores=NC, num_subcores=NS)

  @functools.partial(
      pl.kernel,
      out_type=jax.ShapeDtypeStruct((NC, N_NODES, F), jnp.float32),
      mesh=mesh,
      compiler_params=pltpu.CompilerParams(use_tc_tiling_on_sc=(F % 128 == 0)),
      scratch_types=[
          pltpu.VMEM((4, 2, BATCH), jnp.int32),
          pltpu.VMEM((BATCH, F), jnp.float32),
          pltpu.VMEM((BATCH, F), jnp.float32),
          pltpu.VMEM((BATCH, F), jnp.float32),
          pltpu.VMEM_SHARED((N_NODES, F), jnp.float32),
          pltpu.SemaphoreType.DMA,
          pltpu.SemaphoreType.DMA,
          pltpu.SemaphoreType.DMA,
      ],
  )
  def seg_sum(x_hbm, ei_hbm, zeros_hbm, out_hbm,
              idxr, rows0, rows1, rows2, acc,
              isem, gsem, ssem):
    c = lax.axis_index("c")
    s = lax.axis_index("s")
    wid = c * NS + s
    stripe_off = pl.multiple_of(s * ROWS_A, 8)
    rows = (rows0, rows1, rows2)

    def start_idx(j, b4):
      off = pl.multiple_of((j * NW + wid) * BATCH, BATCH)
      pltpu.async_copy(ei_hbm.at[:, pl.ds(off, BATCH)], idxr.at[b4], isem)

    def wait_idx(b4):
      pltpu.make_async_copy(ei_hbm.at[:, pl.ds(0, BATCH)], idxr.at[b4],
                            isem).wait()

    def start_gather(b3, b4):
      pltpu.async_copy(x_hbm.at[idxr.at[b4, 0]], rows[b3], gsem)

    def wait_gather(b3, b4):
      pltpu.make_async_copy(x_hbm.at[idxr.at[b4, 0]], rows[b3], gsem).wait()

    def start_scatter(b3, b4):
      pltpu.async_copy(rows[b3], acc.at[idxr.at[b4, 1]], ssem, add=True)

    def wait_scatter(b3, b4):
      pltpu.make_async_copy(rows[b3], acc.at[idxr.at[b4, 1]], ssem).wait()

    start_idx(0, 0)

    @pl.when(s < NS - 1)
    def _():
      pltpu.async_copy(zeros_hbm.at[pl.ds(stripe_off, ROWS_A)],
                       acc.at[pl.ds(stripe_off, ROWS_A)], ssem)

    @pl.when(s == NS - 1)
    def _():
      pltpu.async_copy(zeros_hbm.at[pl.ds((NS - 1) * ROWS_A, ROWS_B)],
                       acc.at[pl.ds((NS - 1) * ROWS_A, ROWS_B)], ssem)

    def body_steady(j, prefetch):
      wait_scatter((j - 3) % 3, (j - 3) % 4)
      if prefetch:
        start_idx(j + 1, (j + 1) % 4)
      wait_idx(j % 4)
      start_gather(j % 3, j % 4)
      wait_gather((j - 1) % 3, (j - 1) % 4)
      start_scatter((j - 1) % 3, (j - 1) % 4)

    wait_idx(0)
    start_gather(0, 0)
    start_idx(1, 1)
    wait_idx(1)
    start_gather(1, 1)
    start_idx(2, 2)

    @pl.when(s < NS - 1)
    def _():
      pltpu.make_async_copy(zeros_hbm.at[pl.ds(stripe_off, ROWS_A)],
                            acc.at[pl.ds(stripe_off, ROWS_A)], ssem).wait()

    @pl.when(s == NS - 1)
    def _():
      pltpu.make_async_copy(zeros_hbm.at[pl.ds((NS - 1) * ROWS_A, ROWS_B)],
                            acc.at[pl.ds((NS - 1) * ROWS_A, ROWS_B)],
                            ssem).wait()

    plsc.subcore_barrier()

    wait_gather(0, 0)
    start_scatter(0, 0)
    wait_idx(2)
    start_gather(2, 2)
    start_idx(3, 3)
    wait_gather(1, 1)
    start_scatter(1, 1)

    def body_steady_static(t, j_dyn):
      ts = t + 3
      wait_scatter((ts - 3) % 3, (ts - 3) % 4)
      start_idx(j_dyn + 1, (ts + 1) % 4)
      wait_idx(ts % 4)
      start_gather(ts % 3, ts % 4)
      wait_gather((ts - 1) % 3, (ts - 1) % 4)
      start_scatter((ts - 1) % 3, (ts - 1) % 4)

    def body(g, carry):
      for t in range(12):
        j = 12 * g + 3 + t
        body_steady_static(t, j)
      return carry

    lax.fori_loop(0, (NB_MAIN - 6) // 12, body, 0)

    for j in range(NB_MAIN - 3, NB_MAIN):
      body_steady(j, j + 1 <= NB_MAIN - 1)

    wait_gather((NB_MAIN - 1) % 3, (NB_MAIN - 1) % 4)
    start_scatter((NB_MAIN - 1) % 3, (NB_MAIN - 1) % 4)
    wait_scatter((NB_MAIN - 3) % 3, (NB_MAIN - 3) % 4)
    wait_scatter((NB_MAIN - 2) % 3, (NB_MAIN - 2) % 4)
    wait_scatter((NB_MAIN - 1) % 3, (NB_MAIN - 1) % 4)

    @pl.when(wid < NB_EXTRA)
    def _():
      eoff = pl.multiple_of((NB_MAIN * NW + wid) * BATCH, BATCH)
      pltpu.sync_copy(ei_hbm.at[:, pl.ds(eoff, BATCH)], idxr.at[0])
      pltpu.async_copy(x_hbm.at[idxr.at[0, 0]], rows[0], gsem).wait()
      pltpu.async_copy(rows[0], acc.at[idxr.at[0, 1]], ssem, add=True).wait()

    plsc.subcore_barrier()

    @pl.when(s < NS - 1)
    def _():
      pltpu.sync_copy(acc.at[pl.ds(stripe_off, ROWS_A)],
                      out_hbm.at[c, pl.ds(stripe_off, ROWS_A)])

    @pl.when(s == NS - 1)
    def _():
      pltpu.sync_copy(acc.at[pl.ds((NS - 1) * ROWS_A, ROWS_B)],
                      out_hbm.at[c, pl.ds((NS - 1) * ROWS_A, ROWS_B)])

  return seg_sum


_make_seg_sum = functools.lru_cache(None)(_make_seg_sum)

_BLK = 2000


def _tc_base_body(x_ref, ws_ref, b_ref, o_ref):
  o_ref[...] = (jnp.dot(x_ref[...], ws_ref[...],
                        preferred_element_type=jnp.float32) + b_ref[...])


def _tc_base(x, Ws, b):
  n, f_in = x.shape
  f_out = Ws.shape[1]
  grid = n // _BLK
  return pl.pallas_call(
      _tc_base_body,
      grid=(grid,),
      in_specs=[
          pl.BlockSpec((_BLK, f_in), lambda i: (i, 0)),
          pl.BlockSpec((f_in, f_out), lambda i: (0, 0)),
          pl.BlockSpec((1, f_out), lambda i: (0, 0)),
      ],
      out_specs=pl.BlockSpec((_BLK, f_out), lambda i: (i, 0)),
      out_shape=jax.ShapeDtypeStruct((n, f_out), jnp.float32),
  )(x, Ws, b.reshape(1, f_out))


def _tc_layer_body(p_ref, x_ref, w_ref, ws_ref, b_ref, o_ref):
  agg = p_ref[0] + p_ref[1]
  o_ref[...] = (jnp.dot(agg, w_ref[...], preferred_element_type=jnp.float32)
                + jnp.dot(x_ref[...], ws_ref[...],
                          preferred_element_type=jnp.float32)
                + b_ref[...])


def _tc_layer(p, x, W, Ws, b):
  n, f_in = x.shape
  f_out = W.shape[1]
  grid = n // _BLK
  return pl.pallas_call(
      _tc_layer_body,
      grid=(grid,),
      in_specs=[
          pl.BlockSpec((NC, _BLK, f_in), lambda i: (0, i, 0)),
          pl.BlockSpec((_BLK, f_in), lambda i: (i, 0)),
          pl.BlockSpec((f_in, f_out), lambda i: (0, 0)),
          pl.BlockSpec((f_in, f_out), lambda i: (0, 0)),
          pl.BlockSpec((1, f_out), lambda i: (0, 0)),
      ],
      out_specs=pl.BlockSpec((_BLK, f_out), lambda i: (i, 0)),
      out_shape=jax.ShapeDtypeStruct((n, f_out), jnp.float32),
  )(p, x, W, Ws, b.reshape(1, f_out))


def _tc_layer2_body(p_ref, x_ref, w_ref, ws_ref, b_ref,
                    wo_ref, wso_ref, bo_ref, sup_ref, base_ref):
  agg = p_ref[0] + p_ref[1]
  x2 = (jnp.dot(agg, w_ref[...], preferred_element_type=jnp.float32)
        + jnp.dot(x_ref[...], ws_ref[...], preferred_element_type=jnp.float32)
        + b_ref[...])
  sup_ref[...] = jnp.dot(x2, wo_ref[...], preferred_element_type=jnp.float32)
  base_ref[...] = (jnp.dot(x2, wso_ref[...],
                           preferred_element_type=jnp.float32) + bo_ref[...])


def _tc_layer2(p, x, W, Ws, b, Wo, Wso, bo):
  n, f_in = x.shape
  f_mid = W.shape[1]
  f_out = Wo.shape[1]
  grid = n // _BLK
  return pl.pallas_call(
      _tc_layer2_body,
      grid=(grid,),
      in_specs=[
          pl.BlockSpec((NC, _BLK, f_in), lambda i: (0, i, 0)),
          pl.BlockSpec((_BLK, f_in), lambda i: (i, 0)),
          pl.BlockSpec((f_in, f_mid), lambda i: (0, 0)),
          pl.BlockSpec((f_in, f_mid), lambda i: (0, 0)),
          pl.BlockSpec((1, f_mid), lambda i: (0, 0)),
          pl.BlockSpec((f_mid, f_out), lambda i: (0, 0)),
          pl.BlockSpec((f_mid, f_out), lambda i: (0, 0)),
          pl.BlockSpec((1, f_out), lambda i: (0, 0)),
      ],
      out_specs=[
          pl.BlockSpec((_BLK, f_out), lambda i: (i, 0)),
          pl.BlockSpec((_BLK, f_out), lambda i: (i, 0)),
      ],
      out_shape=[
          jax.ShapeDtypeStruct((n, f_out), jnp.float32),
          jax.ShapeDtypeStruct((n, f_out), jnp.float32),
      ],
  )(p, x, W, Ws, b.reshape(1, f_mid), Wo, Wso, bo.reshape(1, f_out))


def _tc_comb_body(p_ref, base_ref, w_ref, o_ref):
  agg = p_ref[0] + p_ref[1]
  o_ref[...] = (jnp.dot(agg, w_ref[...], preferred_element_type=jnp.float32)
                + base_ref[...])


def _tc_comb(p, base, W):
  n, f_out = base.shape
  f_in = W.shape[0]
  grid = n // _BLK
  return pl.pallas_call(
      _tc_comb_body,
      grid=(grid,),
      in_specs=[
          pl.BlockSpec((NC, _BLK, f_in), lambda i: (0, i, 0)),
          pl.BlockSpec((_BLK, f_out), lambda i: (i, 0)),
          pl.BlockSpec((f_in, f_out), lambda i: (0, 0)),
      ],
      out_specs=pl.BlockSpec((_BLK, f_out), lambda i: (i, 0)),
      out_shape=jax.ShapeDtypeStruct((n, f_out), jnp.float32),
  )(p, base, W)


def _tc_comb2_body(p_ref, base_ref, w_ref, wo_ref, wso_ref, bo_ref,
                   sup_ref, base3_ref):
  agg = p_ref[0] + p_ref[1]
  x2 = (jnp.dot(agg, w_ref[...], preferred_element_type=jnp.float32)
        + base_ref[...])
  sup_ref[...] = jnp.dot(x2, wo_ref[...], preferred_element_type=jnp.float32)
  base3_ref[...] = (jnp.dot(x2, wso_ref[...],
                            preferred_element_type=jnp.float32) + bo_ref[...])


def _tc_comb2(p, base, W, Wo, Wso, bo):
  n, f_mid = base.shape
  f_out = Wo.shape[1]
  grid = n // _BLK
  return pl.pallas_call(
      _tc_comb2_body,
      grid=(grid,),
      in_specs=[
          pl.BlockSpec((NC, _BLK, f_mid), lambda i: (0, i, 0)),
          pl.BlockSpec((_BLK, f_mid), lambda i: (i, 0)),
          pl.BlockSpec((f_mid, f_mid), lambda i: (0, 0)),
          pl.BlockSpec((f_mid, f_out), lambda i: (0, 0)),
          pl.BlockSpec((f_mid, f_out), lambda i: (0, 0)),
          pl.BlockSpec((1, f_out), lambda i: (0, 0)),
      ],
      out_specs=[
          pl.BlockSpec((_BLK, f_out), lambda i: (i, 0)),
          pl.BlockSpec((_BLK, f_out), lambda i: (i, 0)),
      ],
      out_shape=[
          jax.ShapeDtypeStruct((n, f_out), jnp.float32),
          jax.ShapeDtypeStruct((n, f_out), jnp.float32),
      ],
  )(p, base, W, Wo, Wso, bo.reshape(1, f_out))


def _tc_final_body(p_ref, base_ref, o_ref):
  z = p_ref[0] + p_ref[1] + base_ref[...]
  m = jnp.max(z, axis=1, keepdims=True)
  zs = z - m
  o_ref[...] = zs - jnp.log(jnp.sum(jnp.exp(zs), axis=1, keepdims=True))


def _tc_final(p, base):
  n, f_out = base.shape
  grid = n // _BLK
  return pl.pallas_call(
      _tc_final_body,
      grid=(grid,),
      in_specs=[
          pl.BlockSpec((NC, _BLK, f_out), lambda i: (0, i, 0)),
          pl.BlockSpec((_BLK, f_out), lambda i: (i, 0)),
      ],
      out_specs=pl.BlockSpec((_BLK, f_out), lambda i: (i, 0)),
      out_shape=jax.ShapeDtypeStruct((n, f_out), jnp.float32),
  )(p, base)


def kernel(fea, edge_index, W_in, Ws_in, b_in, W_mid, Ws_mid, b_mid,
           W_out, Ws_out, b_out):
  zeros128 = jnp.zeros((N_NODES, 128), jnp.float32)
  zeros40 = jnp.zeros((N_NODES, NCLASS), jnp.float32)
  seg_sum_128 = _make_seg_sum(128)
  seg_sum_40 = _make_seg_sum(NCLASS)

  p1 = seg_sum_128(fea, edge_index, zeros128)
  x1 = _tc_layer(p1, fea, W_in, Ws_in, b_in)
  p2 = seg_sum_128(x1, edge_index, zeros128)
  sup3, base3 = _tc_layer2(p2, x1, W_mid, Ws_mid, b_mid, W_out, Ws_out, b_out)
  p3 = seg_sum_40(sup3, edge_index, zeros40)
  return _tc_final(p3, base3)


# speedup vs baseline: 1.0628x; 1.0122x over previous
import functools

import jax
import jax.numpy as jnp
from jax import lax
from jax.experimental import pallas as pl
from jax.experimental.pallas import tpu as pltpu
from jax.experimental.pallas import tpu_sc as plsc

N_NODES = 10000
N_EDGES = 320000
NCLASS = 40

NC, NS = 2, 16
NW = NC * NS
BATCH = 128
NBATCH = N_EDGES // BATCH
NB_MAIN = NBATCH // NW
NB_EXTRA = NBATCH - NB_MAIN * NW
ROWS_A = 640
ROWS_B = N_NODES - (NS - 1) * ROWS_A


def _make_seg_sum(F):
  mesh = plsc.VectorSubcoreMesh(core_axis_name="c", subcore_axis_name="s",
                                num_cores=NC, num_subcores=NS)

  @functools.partial(
      pl.kernel,
      out_type=jax.ShapeDtypeStruct((NC, N_NODES, F), jnp.float32),
      mesh=mesh,
      compiler_params=pltpu.CompilerParams(use_tc_tiling_on_sc=(F % 128 == 0)),
      scratch_types=[
          pltpu.VMEM((4, 2, BATCH), jnp.int32),
          pltpu.VMEM((BATCH, F), jnp.float32),
          pltpu.VMEM((BATCH, F), jnp.float32),
          pltpu.VMEM((BATCH, F), jnp.float32),
          pltpu.VMEM_SHARED((N_NODES, F), jnp.float32),
          pltpu.SemaphoreType.DMA,
          pltpu.SemaphoreType.DMA,
          pltpu.SemaphoreType.DMA,
      ],
  )
  def seg_sum(x_hbm, ei_hbm, zeros_hbm, out_hbm,
              idxr, rows0, rows1, rows2, acc,
              isem, gsem, ssem):
    c = lax.axis_index("c")
    s = lax.axis_index("s")
    wid = c * NS + s
    stripe_off = pl.multiple_of(s * ROWS_A, 8)
    rows = (rows0, rows1, rows2)

    def start_idx(j, b4):
      off = pl.multiple_of((j * NW + wid) * BATCH, BATCH)
      pltpu.async_copy(ei_hbm.at[:, pl.ds(off, BATCH)], idxr.at[b4], isem)

    def wait_idx(b4):
      pltpu.make_async_copy(ei_hbm.at[:, pl.ds(0, BATCH)], idxr.at[b4],
                            isem).wait()

    def start_gather(b3, b4):
      pltpu.async_copy(x_hbm.at[idxr.at[b4, 0]], rows[b3], gsem)

    def wait_gather(b3, b4):
      pltpu.make_async_copy(x_hbm.at[idxr.at[b4, 0]], rows[b3], gsem).wait()

    def start_scatter(b3, b4):
      pltpu.async_copy(rows[b3], acc.at[idxr.at[b4, 1]], ssem, add=True)

    def wait_scatter(b3, b4):
      pltpu.make_async_copy(rows[b3], acc.at[idxr.at[b4, 1]], ssem).wait()

    start_idx(0, 0)

    @pl.when(s < NS - 1)
    def _():
      pltpu.async_copy(zeros_hbm.at[pl.ds(stripe_off, ROWS_A)],
                       acc.at[pl.ds(stripe_off, ROWS_A)], ssem)

    @pl.when(s == NS - 1)
    def _():
      pltpu.async_copy(zeros_hbm.at[pl.ds((NS - 1) * ROWS_A, ROWS_B)],
                       acc.at[pl.ds((NS - 1) * ROWS_A, ROWS_B)], ssem)

    def body_steady(j, prefetch):
      wait_scatter((j - 3) % 3, (j - 3) % 4)
      if prefetch:
        start_idx(j + 1, (j + 1) % 4)
      wait_idx(j % 4)
      start_gather(j % 3, j % 4)
      wait_gather((j - 1) % 3, (j - 1) % 4)
      start_scatter((j - 1) % 3, (j - 1) % 4)

    wait_idx(0)
    start_gather(0, 0)
    start_idx(1, 1)
    wait_idx(1)
    start_gather(1, 1)
    start_idx(2, 2)

    @pl.when(s < NS - 1)
    def _():
      pltpu.make_async_copy(zeros_hbm.at[pl.ds(stripe_off, ROWS_A)],
                            acc.at[pl.ds(stripe_off, ROWS_A)], ssem).wait()

    @pl.when(s == NS - 1)
    def _():
      pltpu.make_async_copy(zeros_hbm.at[pl.ds((NS - 1) * ROWS_A, ROWS_B)],
                            acc.at[pl.ds((NS - 1) * ROWS_A, ROWS_B)],
                            ssem).wait()

    plsc.subcore_barrier()

    wait_gather(0, 0)
    start_scatter(0, 0)
    wait_idx(2)
    start_gather(2, 2)
    start_idx(3, 3)
    wait_gather(1, 1)
    start_scatter(1, 1)

    def body_steady_static(t, j_dyn):
      ts = t + 3
      wait_scatter((ts - 3) % 3, (ts - 3) % 4)
      start_idx(j_dyn + 1, (ts + 1) % 4)
      wait_idx(ts % 4)
      start_gather(ts % 3, ts % 4)
      wait_gather((ts - 1) % 3, (ts - 1) % 4)
      start_scatter((ts - 1) % 3, (ts - 1) % 4)

    def body(g, carry):
      for t in range(12):
        j = 12 * g + 3 + t
        body_steady_static(t, j)
      return carry

    lax.fori_loop(0, (NB_MAIN - 6) // 12, body, 0)

    for j in range(NB_MAIN - 3, NB_MAIN):
      body_steady(j, j + 1 <= NB_MAIN - 1)

    wait_gather((NB_MAIN - 1) % 3, (NB_MAIN - 1) % 4)
    start_scatter((NB_MAIN - 1) % 3, (NB_MAIN - 1) % 4)
    wait_scatter((NB_MAIN - 3) % 3, (NB_MAIN - 3) % 4)
    wait_scatter((NB_MAIN - 2) % 3, (NB_MAIN - 2) % 4)
    wait_scatter((NB_MAIN - 1) % 3, (NB_MAIN - 1) % 4)

    @pl.when(wid < NB_EXTRA)
    def _():
      eoff = pl.multiple_of((NB_MAIN * NW + wid) * BATCH, BATCH)
      pltpu.sync_copy(ei_hbm.at[:, pl.ds(eoff, BATCH)], idxr.at[0])
      pltpu.async_copy(x_hbm.at[idxr.at[0, 0]], rows[0], gsem).wait()
      pltpu.async_copy(rows[0], acc.at[idxr.at[0, 1]], ssem, add=True).wait()

    plsc.subcore_barrier()

    @pl.when(s < NS - 1)
    def _():
      pltpu.sync_copy(acc.at[pl.ds(stripe_off, ROWS_A)],
                      out_hbm.at[c, pl.ds(stripe_off, ROWS_A)])

    @pl.when(s == NS - 1)
    def _():
      pltpu.sync_copy(acc.at[pl.ds((NS - 1) * ROWS_A, ROWS_B)],
                      out_hbm.at[c, pl.ds((NS - 1) * ROWS_A, ROWS_B)])

  return seg_sum


_make_seg_sum = functools.lru_cache(None)(_make_seg_sum)

_BLK = 5000


def _tc_base_body(x_ref, ws_ref, b_ref, o_ref):
  o_ref[...] = (jnp.dot(x_ref[...], ws_ref[...],
                        preferred_element_type=jnp.float32) + b_ref[...])


def _tc_base(x, Ws, b):
  n, f_in = x.shape
  f_out = Ws.shape[1]
  grid = n // _BLK
  return pl.pallas_call(
      _tc_base_body,
      grid=(grid,),
      in_specs=[
          pl.BlockSpec((_BLK, f_in), lambda i: (i, 0)),
          pl.BlockSpec((f_in, f_out), lambda i: (0, 0)),
          pl.BlockSpec((1, f_out), lambda i: (0, 0)),
      ],
      out_specs=pl.BlockSpec((_BLK, f_out), lambda i: (i, 0)),
      out_shape=jax.ShapeDtypeStruct((n, f_out), jnp.float32),
  )(x, Ws, b.reshape(1, f_out))


def _tc_layer_body(p_ref, x_ref, w_ref, ws_ref, b_ref, o_ref):
  agg = p_ref[0] + p_ref[1]
  o_ref[...] = (jnp.dot(agg, w_ref[...], preferred_element_type=jnp.float32)
                + jnp.dot(x_ref[...], ws_ref[...],
                          preferred_element_type=jnp.float32)
                + b_ref[...])


def _tc_layer(p, x, W, Ws, b):
  n, f_in = x.shape
  f_out = W.shape[1]
  grid = n // _BLK
  return pl.pallas_call(
      _tc_layer_body,
      grid=(grid,),
      in_specs=[
          pl.BlockSpec((NC, _BLK, f_in), lambda i: (0, i, 0)),
          pl.BlockSpec((_BLK, f_in), lambda i: (i, 0)),
          pl.BlockSpec((f_in, f_out), lambda i: (0, 0)),
          pl.BlockSpec((f_in, f_out), lambda i: (0, 0)),
          pl.BlockSpec((1, f_out), lambda i: (0, 0)),
      ],
      out_specs=pl.BlockSpec((_BLK, f_out), lambda i: (i, 0)),
      out_shape=jax.ShapeDtypeStruct((n, f_out), jnp.float32),
  )(p, x, W, Ws, b.reshape(1, f_out))


def _tc_layer2_body(p_ref, x_ref, w_ref, ws_ref, b_ref,
                    wo_ref, wso_ref, bo_ref, sup_ref, base_ref):
  agg = p_ref[0] + p_ref[1]
  x2 = (jnp.dot(agg, w_ref[...], preferred_element_type=jnp.float32)
        + jnp.dot(x_ref[...], ws_ref[...], preferred_element_type=jnp.float32)
        + b_ref[...])
  sup_ref[...] = jnp.dot(x2, wo_ref[...], preferred_element_type=jnp.float32)
  base_ref[...] = (jnp.dot(x2, wso_ref[...],
                           preferred_element_type=jnp.float32) + bo_ref[...])


def _tc_layer2(p, x, W, Ws, b, Wo, Wso, bo):
  n, f_in = x.shape
  f_mid = W.shape[1]
  f_out = Wo.shape[1]
  grid = n // _BLK
  return pl.pallas_call(
      _tc_layer2_body,
      grid=(grid,),
      in_specs=[
          pl.BlockSpec((NC, _BLK, f_in), lambda i: (0, i, 0)),
          pl.BlockSpec((_BLK, f_in), lambda i: (i, 0)),
          pl.BlockSpec((f_in, f_mid), lambda i: (0, 0)),
          pl.BlockSpec((f_in, f_mid), lambda i: (0, 0)),
          pl.BlockSpec((1, f_mid), lambda i: (0, 0)),
          pl.BlockSpec((f_mid, f_out), lambda i: (0, 0)),
          pl.BlockSpec((f_mid, f_out), lambda i: (0, 0)),
          pl.BlockSpec((1, f_out), lambda i: (0, 0)),
      ],
      out_specs=[
          pl.BlockSpec((_BLK, f_out), lambda i: (i, 0)),
          pl.BlockSpec((_BLK, f_out), lambda i: (i, 0)),
      ],
      out_shape=[
          jax.ShapeDtypeStruct((n, f_out), jnp.float32),
          jax.ShapeDtypeStruct((n, f_out), jnp.float32),
      ],
  )(p, x, W, Ws, b.reshape(1, f_mid), Wo, Wso, bo.reshape(1, f_out))


def _tc_comb_body(p_ref, base_ref, w_ref, o_ref):
  agg = p_ref[0] + p_ref[1]
  o_ref[...] = (jnp.dot(agg, w_ref[...], preferred_element_type=jnp.float32)
                + base_ref[...])


def _tc_comb(p, base, W):
  n, f_out = base.shape
  f_in = W.shape[0]
  grid = n // _BLK
  return pl.pallas_call(
      _tc_comb_body,
      grid=(grid,),
      in_specs=[
          pl.BlockSpec((NC, _BLK, f_in), lambda i: (0, i, 0)),
          pl.BlockSpec((_BLK, f_out), lambda i: (i, 0)),
          pl.BlockSpec((f_in, f_out), lambda i: (0, 0)),
      ],
      out_specs=pl.BlockSpec((_BLK, f_out), lambda i: (i, 0)),
      out_shape=jax.ShapeDtypeStruct((n, f_out), jnp.float32),
  )(p, base, W)


def _tc_comb2_body(p_ref, base_ref, w_ref, wo_ref, wso_ref, bo_ref,
                   sup_ref, base3_ref):
  agg = p_ref[0] + p_ref[1]
  x2 = (jnp.dot(agg, w_ref[...], preferred_element_type=jnp.float32)
        + base_ref[...])
  sup_ref[...] = jnp.dot(x2, wo_ref[...], preferred_element_type=jnp.float32)
  base3_ref[...] = (jnp.dot(x2, wso_ref[...],
                            preferred_element_type=jnp.float32) + bo_ref[...])


def _tc_comb2(p, base, W, Wo, Wso, bo):
  n, f_mid = base.shape
  f_out = Wo.shape[1]
  grid = n // _BLK
  return pl.pallas_call(
      _tc_comb2_body,
      grid=(grid,),
      in_specs=[
          pl.BlockSpec((NC, _BLK, f_mid), lambda i: (0, i, 0)),
          pl.BlockSpec((_BLK, f_mid), lambda i: (i, 0)),
          pl.BlockSpec((f_mid, f_mid), lambda i: (0, 0)),
          pl.BlockSpec((f_mid, f_out), lambda i: (0, 0)),
          pl.BlockSpec((f_mid, f_out), lambda i: (0, 0)),
          pl.BlockSpec((1, f_out), lambda i: (0, 0)),
      ],
      out_specs=[
          pl.BlockSpec((_BLK, f_out), lambda i: (i, 0)),
          pl.BlockSpec((_BLK, f_out), lambda i: (i, 0)),
      ],
      out_shape=[
          jax.ShapeDtypeStruct((n, f_out), jnp.float32),
          jax.ShapeDtypeStruct((n, f_out), jnp.float32),
      ],
  )(p, base, W, Wo, Wso, bo.reshape(1, f_out))


def _tc_final_body(p_ref, base_ref, o_ref):
  z = p_ref[0] + p_ref[1] + base_ref[...]
  m = jnp.max(z, axis=1, keepdims=True)
  zs = z - m
  o_ref[...] = zs - jnp.log(jnp.sum(jnp.exp(zs), axis=1, keepdims=True))


def _tc_final(p, base):
  n, f_out = base.shape
  grid = n // _BLK
  return pl.pallas_call(
      _tc_final_body,
      grid=(grid,),
      in_specs=[
          pl.BlockSpec((NC, _BLK, f_out), lambda i: (0, i, 0)),
          pl.BlockSpec((_BLK, f_out), lambda i: (i, 0)),
      ],
      out_specs=pl.BlockSpec((_BLK, f_out), lambda i: (i, 0)),
      out_shape=jax.ShapeDtypeStruct((n, f_out), jnp.float32),
  )(p, base)


def kernel(fea, edge_index, W_in, Ws_in, b_in, W_mid, Ws_mid, b_mid,
           W_out, Ws_out, b_out):
  zeros128 = jnp.zeros((N_NODES, 128), jnp.float32)
  zeros40 = jnp.zeros((N_NODES, NCLASS), jnp.float32)
  seg_sum_128 = _make_seg_sum(128)
  seg_sum_40 = _make_seg_sum(NCLASS)

  p1 = seg_sum_128(fea, edge_index, zeros128)
  x1 = _tc_layer(p1, fea, W_in, Ws_in, b_in)
  p2 = seg_sum_128(x1, edge_index, zeros128)
  sup3, base3 = _tc_layer2(p2, x1, W_mid, Ws_mid, b_mid, W_out, Ws_out, b_out)
  p3 = seg_sum_40(sup3, edge_index, zeros40)
  return _tc_final(p3, base3)
